# Initial kernel scaffold; baseline (speedup 1.0000x reference)
import jax
import jax.numpy as jnp
from jax.experimental import pallas as pl


def kernel(x, gate_W, gate_b, W1, b1, W2, b2):
    raise NotImplementedError("write your pallas kernel here")


# speedup vs baseline: 2.2858x; 2.2858x over previous
import functools
import math

import jax
import jax.numpy as jnp
from jax import lax
from jax.experimental import pallas as pl
from jax.experimental.pallas import tpu as pltpu

D_MODEL = 1024
DIM_FF = 4096
N_EXPERTS = 8
N_TOK = 2048
BLK = 256
TB = N_TOK // BLK
FF_BLK = 2048
NF = DIM_FF // FF_BLK


def _gate_weights(logits):
    m = jnp.max(logits, axis=-1, keepdims=True)
    p = jnp.exp(logits - m)
    w = p / jnp.sum(p, axis=-1, keepdims=True)
    iot = lax.broadcasted_iota(jnp.int32, w.shape, 1)
    w1v = jnp.max(w, axis=-1, keepdims=True)
    i1 = jnp.min(jnp.where(w == w1v, iot, N_EXPERTS), axis=-1, keepdims=True)
    wm = jnp.where(iot == i1, -jnp.inf, w)
    w2v = jnp.max(wm, axis=-1, keepdims=True)
    i2 = jnp.min(jnp.where(wm == w2v, iot, N_EXPERTS), axis=-1, keepdims=True)
    keep = (iot == i1) | (iot == i2)
    wk = jnp.where(keep, w, 0.0)
    return wk / (jnp.sum(wk, axis=-1, keepdims=True) + 1e-9)


def _dense_body(x_ref, gw_ref, gb_ref, w1_ref, b1_ref, w2_ref, b2_ref,
                out_ref, gate_sc, acc_sc):
    e = pl.program_id(0)
    f = pl.program_id(1)
    tb = pl.program_id(2)
    rows = pl.ds(tb * BLK, BLK)

    @pl.when((e == 0) & (f == 0))
    def _():
        logits = jnp.dot(x_ref[...], gw_ref[...],
                         preferred_element_type=jnp.float32) + gb_ref[...]
        gate_sc[rows, :] = _gate_weights(logits)

    h = jnp.dot(x_ref[...], w1_ref[0], preferred_element_type=jnp.float32)
    h = h + b1_ref[0]
    h = 0.5 * h * (1.0 + lax.erf(h * (1.0 / math.sqrt(2.0))))
    eo = jnp.dot(h, w2_ref[0], preferred_element_type=jnp.float32)

    @pl.when(f == 0)
    def _():
        eo2 = eo + b2_ref[0]
        gw_blk = gate_sc[rows, :]
        col = lax.broadcasted_iota(jnp.int32, (BLK, N_EXPERTS), 1)
        w_e = jnp.sum(jnp.where(col == e, gw_blk, 0.0), axis=1, keepdims=True)
        contrib = eo2 * w_e

        @pl.when(e == 0)
        def _():
            acc_sc[rows, :] = contrib

        @pl.when(e > 0)
        def _():
            acc_sc[rows, :] = acc_sc[rows, :] + contrib

    @pl.when(f > 0)
    def _():
        gw_blk = gate_sc[rows, :]
        col = lax.broadcasted_iota(jnp.int32, (BLK, N_EXPERTS), 1)
        w_e = jnp.sum(jnp.where(col == e, gw_blk, 0.0), axis=1, keepdims=True)
        acc_sc[rows, :] = acc_sc[rows, :] + eo * w_e

    @pl.when((e == N_EXPERTS - 1) & (f == NF - 1))
    def _():
        out_ref[...] = acc_sc[rows, :]


def kernel(x, gate_W, gate_b, W1, b1, W2, b2):
    b1 = b1.reshape(N_EXPERTS, 1, DIM_FF)
    b2 = b2.reshape(N_EXPERTS, 1, D_MODEL)
    grid = (N_EXPERTS, NF, TB)
    return pl.pallas_call(
        _dense_body,
        grid=grid,
        in_specs=[
            pl.BlockSpec((BLK, D_MODEL), lambda e, f, tb: (tb, 0)),
            pl.BlockSpec((D_MODEL, N_EXPERTS), lambda e, f, tb: (0, 0)),
            pl.BlockSpec((N_EXPERTS,), lambda e, f, tb: (0,)),
            pl.BlockSpec((1, D_MODEL, FF_BLK), lambda e, f, tb: (e, 0, f)),
            pl.BlockSpec((1, 1, FF_BLK), lambda e, f, tb: (e, 0, f)),
            pl.BlockSpec((1, FF_BLK, D_MODEL), lambda e, f, tb: (e, f, 0)),
            pl.BlockSpec((1, 1, D_MODEL), lambda e, f, tb: (e, 0, 0)),
        ],
        out_specs=pl.BlockSpec((BLK, D_MODEL), lambda e, f, tb: (tb, 0)),
        out_shape=jax.ShapeDtypeStruct((N_TOK, D_MODEL), jnp.float32),
        scratch_shapes=[
            pltpu.VMEM((N_TOK, N_EXPERTS), jnp.float32),
            pltpu.VMEM((N_TOK, D_MODEL), jnp.float32),
        ],
    )(x, gate_W, gate_b, W1, b1, W2, b2)
